# Initial kernel scaffold; baseline (speedup 1.0000x reference)
import jax
import jax.numpy as jnp
from jax.experimental import pallas as pl


def kernel(x, edge_index, edge_weight, W1, W2, W_out):
    raise NotImplementedError("write your pallas kernel here")


# speedup vs baseline: 5.7054x; 5.7054x over previous
import functools

import jax
import jax.numpy as jnp
from jax import lax
from jax.experimental import pallas as pl
from jax.experimental.pallas import tpu as pltpu
from jax.experimental.pallas import tpu_sc as plsc

N_NODES = 10000
N_EDGES = 320000
D_FEAT = 128
H_DIM = 256
N_LABELS = 64

NC = 2
NS = 16
LANES = 16
GB = 80
NPAD = 10240
ROWS_PER_SUB = NPAD // NS
ZCH = 128

_sc_mesh = plsc.VectorSubcoreMesh(core_axis_name="c", subcore_axis_name="s")


def _zero_fill(ref, n_rows, n_feat):
  z = jnp.zeros((LANES,), jnp.float32)

  def body(r, carry):
    for f in range(n_feat // LANES):
      ref[r, pl.ds(f * LANES, LANES)] = z
    return carry

  lax.fori_loop(0, n_rows, body, None)


def _fill_const(ref, n_rows, n_feat, val):
  v = jnp.full((LANES,), val, jnp.float32)

  def body(r, carry):
    for f in range(n_feat // LANES):
      ref[r, pl.ds(f * LANES, LANES)] = v
    return carry

  lax.fori_loop(0, n_rows, body, None)


DEGW = 128


def _deg_body(row_hbm, out_hbm, idx_v, ones_v, zb_v, acc_sh):
  cid = lax.axis_index("c")
  sid = lax.axis_index("s")

  _zero_fill(zb_v, ZCH, DEGW)
  _fill_const(ones_v, GB, DEGW, 1.0)
  for z in range(ROWS_PER_SUB // ZCH):
    pltpu.sync_copy(zb_v, acc_sh.at[pl.ds(sid * ROWS_PER_SUB + z * ZCH, ZCH)])
  plsc.subcore_barrier()

  edges_per_sub = N_EDGES // (NC * NS)
  base = (cid * NS + sid) * edges_per_sub

  def body(b, carry):
    pltpu.sync_copy(row_hbm.at[pl.ds(base + b * GB, GB)], idx_v)
    pltpu.sync_copy(ones_v, acc_sh.at[idx_v], add=True)
    return carry

  lax.fori_loop(0, edges_per_sub // GB, body, None)
  plsc.subcore_barrier()

  pltpu.sync_copy(
      acc_sh.at[pl.ds(sid * ROWS_PER_SUB, ROWS_PER_SUB)],
      out_hbm.at[pl.ds(cid * NPAD + sid * ROWS_PER_SUB, ROWS_PER_SUB)])


_deg_call = pl.kernel(
    _deg_body,
    out_type=jax.ShapeDtypeStruct((NC * NPAD, DEGW), jnp.float32),
    mesh=_sc_mesh,
    scratch_types=[
        pltpu.VMEM((GB,), jnp.int32),
        pltpu.VMEM((GB, DEGW), jnp.float32),
        pltpu.VMEM((ZCH, DEGW), jnp.float32),
        pltpu.VMEM_SHARED((NPAD, DEGW), jnp.float32),
    ],
)


def _agg_body(feat_split, y_hbm, row_hbm, col_hbm, ew_hbm, out_hbm,
              idx_v, col_v, ew_v, rows_v, zb_v, acc_sh, sem):
  dc = 128
  cid = lax.axis_index("c")
  sid = lax.axis_index("s")
  fch = dc // LANES

  _zero_fill(zb_v, ZCH, dc)
  for z in range(ROWS_PER_SUB // ZCH):
    pltpu.sync_copy(zb_v, acc_sh.at[pl.ds(sid * ROWS_PER_SUB + z * ZCH, ZCH)])
  plsc.subcore_barrier()

  if feat_split:
    edges_per_sub = N_EDGES // NS
    base = sid * edges_per_sub
    shift = cid * NPAD
  else:
    edges_per_sub = N_EDGES // (NC * NS)
    base = (cid * NS + sid) * edges_per_sub
    shift = 0

  def body(b, carry):
    e0 = base + b * GB
    pltpu.sync_copy(row_hbm.at[pl.ds(e0, GB)], idx_v)
    pltpu.sync_copy(col_hbm.at[pl.ds(e0, GB)], col_v)
    pltpu.sync_copy(ew_hbm.at[pl.ds(e0, GB)], ew_v)
    if feat_split:
      for f in range(GB // LANES):
        sl = pl.ds(f * LANES, LANES)
        idx_v[sl] = idx_v[sl] + shift
    pltpu.async_copy(y_hbm.at[idx_v], rows_v, sem).wait()

    def scale(c, carry2):
      wv = ew_v[pl.ds(c * LANES, LANES)]
      for l in range(LANES):
        w = wv[l]
        g = c * LANES + l
        for f in range(fch):
          sl = pl.ds(f * LANES, LANES)
          rows_v[g, sl] = rows_v[g, sl] * w
      return carry2

    lax.fori_loop(0, GB // LANES, scale, None)
    pltpu.sync_copy(rows_v, acc_sh.at[col_v], add=True)
    return carry

  lax.fori_loop(0, edges_per_sub // GB, body, None)
  plsc.subcore_barrier()

  pltpu.sync_copy(
      acc_sh.at[pl.ds(sid * ROWS_PER_SUB, ROWS_PER_SUB)],
      out_hbm.at[pl.ds(cid * NPAD + sid * ROWS_PER_SUB, ROWS_PER_SUB)])


def _make_agg_call(feat_split):
  dc = 128
  return pl.kernel(
      functools.partial(_agg_body, feat_split),
      out_type=jax.ShapeDtypeStruct((NC * NPAD, dc), jnp.float32),
      mesh=_sc_mesh,
      scratch_types=[
          pltpu.VMEM((GB,), jnp.int32),
          pltpu.VMEM((GB,), jnp.int32),
          pltpu.VMEM((GB,), jnp.float32),
          pltpu.VMEM((GB, dc), jnp.float32),
          pltpu.VMEM((ZCH, dc), jnp.float32),
          pltpu.VMEM_SHARED((NPAD, dc), jnp.float32),
          pltpu.SemaphoreType.DMA,
      ],
  )


_agg_edge_split = _make_agg_call(False)
_agg_feat_split = _make_agg_call(True)


BN = 1024
NB = NPAD // BN


def _dis_from(dp0_ref, dp1_ref):
  deg = dp0_ref[:, 0] + dp1_ref[:, 0]
  return jnp.where(deg > 0, lax.rsqrt(deg), 0.0)


def _prep_body(dp0_ref, dp1_ref, x_ref, y_ref):
  dis = _dis_from(dp0_ref, dp1_ref)
  y_ref[...] = dis[:, None] * x_ref[...]


_prep_call = pl.pallas_call(
    _prep_body,
    grid=(NB,),
    in_specs=[
        pl.BlockSpec((BN, DEGW), lambda i: (i, 0)),
        pl.BlockSpec((BN, DEGW), lambda i: (i, 0)),
        pl.BlockSpec((BN, D_FEAT), lambda i: (i, 0)),
    ],
    out_specs=pl.BlockSpec((BN, D_FEAT), lambda i: (i, 0)),
    out_shape=jax.ShapeDtypeStruct((NPAD, D_FEAT), jnp.float32),
)


def _mm_t(a, w):
  return lax.dot_general(a, w, (((1,), (1,)), ((), ())),
                         preferred_element_type=jnp.float32)


def _enc1_body(dp0_ref, dp1_ref, x_ref, ag_ref, w1_ref, h_ref, y2_ref):
  dis = _dis_from(dp0_ref, dp1_ref)
  a = dis[:, None] * (ag_ref[0] + ag_ref[1])
  h = _mm_t(x_ref[...], w1_ref[:, :D_FEAT])
  h += _mm_t(a, w1_ref[:, D_FEAT:])
  h = jnp.maximum(h, 0.0)
  h_ref[...] = h
  hh = H_DIM // NC
  y2_ref[0] = dis[:, None] * h[:, :hh]
  y2_ref[1] = dis[:, None] * h[:, hh:]


_enc1_call = pl.pallas_call(
    _enc1_body,
    grid=(NB,),
    in_specs=[
        pl.BlockSpec((BN, DEGW), lambda i: (i, 0)),
        pl.BlockSpec((BN, DEGW), lambda i: (i, 0)),
        pl.BlockSpec((BN, D_FEAT), lambda i: (i, 0)),
        pl.BlockSpec((NC, BN, D_FEAT), lambda i: (0, i, 0)),
        pl.BlockSpec((H_DIM, 2 * D_FEAT), lambda i: (0, 0)),
    ],
    out_specs=[
        pl.BlockSpec((BN, H_DIM), lambda i: (i, 0)),
        pl.BlockSpec((NC, BN, H_DIM // NC), lambda i: (0, i, 0)),
    ],
    out_shape=[
        jax.ShapeDtypeStruct((N_NODES, H_DIM), jnp.float32),
        jax.ShapeDtypeStruct((NC, NPAD, H_DIM // NC), jnp.float32),
    ],
)


def _enc2_body(dp0_ref, dp1_ref, h1_ref, ag_ref, w2_ref, wo_ref, out_ref):
  dis = _dis_from(dp0_ref, dp1_ref)
  hh = H_DIM // NC
  a0 = dis[:, None] * ag_ref[0]
  a1 = dis[:, None] * ag_ref[1]
  h = _mm_t(h1_ref[...], w2_ref[:, :H_DIM])
  h += _mm_t(a0, w2_ref[:, H_DIM:H_DIM + hh])
  h += _mm_t(a1, w2_ref[:, H_DIM + hh:])
  h = jnp.maximum(h, 0.0)
  lg = _mm_t(h, wo_ref[...])
  m = jnp.max(lg, axis=1, keepdims=True)
  s = lg - m
  lse = jnp.log(jnp.sum(jnp.exp(s), axis=1, keepdims=True))
  out_ref[...] = s - lse


_enc2_call = pl.pallas_call(
    _enc2_body,
    grid=(NB,),
    in_specs=[
        pl.BlockSpec((BN, DEGW), lambda i: (i, 0)),
        pl.BlockSpec((BN, DEGW), lambda i: (i, 0)),
        pl.BlockSpec((BN, H_DIM), lambda i: (i, 0)),
        pl.BlockSpec((NC, BN, H_DIM // NC), lambda i: (0, i, 0)),
        pl.BlockSpec((H_DIM, 2 * H_DIM), lambda i: (0, 0)),
        pl.BlockSpec((N_LABELS, H_DIM), lambda i: (0, 0)),
    ],
    out_specs=pl.BlockSpec((BN, N_LABELS), lambda i: (i, 0)),
    out_shape=jax.ShapeDtypeStruct((N_NODES, N_LABELS), jnp.float32),
)


def kernel(x, edge_index, edge_weight, W1, W2, W_out):
  row = edge_index[0].astype(jnp.int32)
  col = edge_index[1].astype(jnp.int32)
  ew = edge_weight.astype(jnp.float32)

  dp = _deg_call(row)
  dp0 = dp[:N_NODES]
  dp1 = dp[NPAD:NPAD + N_NODES]

  y1 = _prep_call(dp0, dp1, x)
  ag1 = _agg_edge_split(y1, row, col, ew)
  h1, y2 = _enc1_call(dp0, dp1, x,
                      ag1.reshape(NC, NPAD, D_FEAT), W1)
  ag2 = _agg_feat_split(y2.reshape(NC * NPAD, H_DIM // NC), row, col, ew)
  return _enc2_call(dp0, dp1, h1,
                    ag2.reshape(NC, NPAD, H_DIM // NC), W2, W_out)
